# baseline (speedup 1.0000x reference)
import functools

import jax
import jax.numpy as jnp
from jax import lax
from jax.experimental import pallas as pl
from jax.experimental.pallas import tpu as pltpu
from jax.experimental.pallas import tpu_sc as plsc

NUM_TIMESTEPS = 1000
BATCH_SIZE = 4096

_info = plsc.get_sparse_core_info()
_NC, _NS, _L = _info.num_cores, _info.num_subcores, _info.num_lanes
_NW = 4
_BPW = BATCH_SIZE // _NW

_mesh = plsc.VectorSubcoreMesh(
    core_axis_name="c", subcore_axis_name="s", num_cores=1)


@functools.partial(
    pl.kernel,
    mesh=_mesh,
    out_type=(
        jax.ShapeDtypeStruct((BATCH_SIZE,), jnp.float32),
        jax.ShapeDtypeStruct((BATCH_SIZE,), jnp.float32),
    ),
    scratch_types=[
        pltpu.VMEM((_BPW,), jnp.int32),
        pltpu.VMEM((_BPW,), jnp.float32),
        pltpu.VMEM((_BPW,), jnp.float32),
        pltpu.SemaphoreType.DMA,
        pltpu.SemaphoreType.DMA,
    ],
)
def _ddpm_gather(t_hbm, beta_hbm, alpha_hbm, beta_out, alpha_out,
                 t_v, bout_v, aout_v, sem_b, sem_a):
    wid = lax.axis_index("s")
    base = wid * _BPW

    @pl.when(wid < _NW)
    def _():
        pltpu.sync_copy(t_hbm.at[pl.ds(base, _BPW)], t_v)
        cp_b = pltpu.async_copy(beta_hbm.at[t_v], bout_v, sem_b)
        cp_a = pltpu.async_copy(alpha_hbm.at[t_v], aout_v, sem_a)
        cp_b.wait()
        st_b = pltpu.async_copy(bout_v, beta_out.at[pl.ds(base, _BPW)], sem_b)
        cp_a.wait()
        st_a = pltpu.async_copy(aout_v, alpha_out.at[pl.ds(base, _BPW)], sem_a)
        st_b.wait()
        st_a.wait()


def kernel(t, beta, alpha):
    return _ddpm_gather(t.astype(jnp.int32), beta, alpha)


# speedup vs baseline: 1.1012x; 1.1012x over previous
import functools

import jax
import jax.numpy as jnp
from jax import lax
from jax.experimental import pallas as pl
from jax.experimental.pallas import tpu as pltpu
from jax.experimental.pallas import tpu_sc as plsc

NUM_TIMESTEPS = 1000
BATCH_SIZE = 4096

_info = plsc.get_sparse_core_info()
_L = _info.num_lanes
_NW = _info.num_subcores
_BPW = BATCH_SIZE // _NW

_mesh = plsc.VectorSubcoreMesh(
    core_axis_name="c", subcore_axis_name="s", num_cores=1)


@functools.partial(
    pl.kernel,
    mesh=_mesh,
    out_type=(
        jax.ShapeDtypeStruct((BATCH_SIZE,), jnp.float32),
        jax.ShapeDtypeStruct((BATCH_SIZE,), jnp.float32),
    ),
    scratch_types=[
        pltpu.VMEM((_BPW,), jnp.int32),
        pltpu.VMEM((NUM_TIMESTEPS,), jnp.float32),
        pltpu.VMEM((NUM_TIMESTEPS,), jnp.float32),
        pltpu.VMEM((_BPW,), jnp.float32),
        pltpu.VMEM((_BPW,), jnp.float32),
        pltpu.SemaphoreType.DMA,
        pltpu.SemaphoreType.DMA,
        pltpu.SemaphoreType.DMA,
    ],
    compiler_params=pltpu.CompilerParams(needs_layout_passes=False),
)
def _ddpm_gather(t_hbm, beta_hbm, alpha_hbm, beta_out, alpha_out,
                 t_v, beta_v, alpha_v, bout_v, aout_v, sem_t, sem_b, sem_a):
    base = lax.axis_index("s") * _BPW
    ld_t = pltpu.async_copy(t_hbm.at[pl.ds(base, _BPW)], t_v, sem_t)
    ld_b = pltpu.async_copy(beta_hbm, beta_v, sem_b)
    ld_a = pltpu.async_copy(alpha_hbm, alpha_v, sem_a)
    ld_t.wait()
    ld_b.wait()
    ld_a.wait()
    for i in range(_BPW // _L):
        idx = t_v[pl.ds(i * _L, _L)]
        bout_v[pl.ds(i * _L, _L)] = plsc.load_gather(beta_v, [idx])
        aout_v[pl.ds(i * _L, _L)] = plsc.load_gather(alpha_v, [idx])
    st_b = pltpu.async_copy(bout_v, beta_out.at[pl.ds(base, _BPW)], sem_b)
    st_a = pltpu.async_copy(aout_v, alpha_out.at[pl.ds(base, _BPW)], sem_a)
    st_b.wait()
    st_a.wait()


def kernel(t, beta, alpha):
    return _ddpm_gather(t.astype(jnp.int32), beta, alpha)


# speedup vs baseline: 1.1062x; 1.0045x over previous
import functools

import jax
import jax.numpy as jnp
from jax import lax
from jax.experimental import pallas as pl
from jax.experimental.pallas import tpu as pltpu
from jax.experimental.pallas import tpu_sc as plsc

NUM_TIMESTEPS = 1000
BATCH_SIZE = 4096

_info = plsc.get_sparse_core_info()
_L = _info.num_lanes
_NW = _info.num_subcores
_BPW = BATCH_SIZE // _NW

_mesh = plsc.VectorSubcoreMesh(
    core_axis_name="c", subcore_axis_name="s", num_cores=1)


@functools.partial(
    pl.kernel,
    mesh=_mesh,
    out_type=(
        jax.ShapeDtypeStruct((BATCH_SIZE,), jnp.float32),
        jax.ShapeDtypeStruct((BATCH_SIZE,), jnp.float32),
    ),
    scratch_types=[
        pltpu.VMEM((_BPW,), jnp.int32),
        pltpu.VMEM((NUM_TIMESTEPS,), jnp.float32),
        pltpu.VMEM((NUM_TIMESTEPS,), jnp.float32),
        pltpu.VMEM((_BPW,), jnp.float32),
        pltpu.VMEM((_BPW,), jnp.float32),
        pltpu.SemaphoreType.DMA,
        pltpu.SemaphoreType.DMA,
        pltpu.SemaphoreType.DMA,
    ],
    compiler_params=pltpu.CompilerParams(needs_layout_passes=False),
)
def _ddpm_gather(t_hbm, beta_hbm, alpha_hbm, beta_out, alpha_out,
                 t_v, beta_v, alpha_v, bout_v, aout_v, sem_t, sem_b, sem_a):
    base = lax.axis_index("s") * _BPW
    ld_t = pltpu.async_copy(t_hbm.at[pl.ds(base, _BPW)], t_v, sem_t)
    ld_b = pltpu.async_copy(beta_hbm, beta_v, sem_b)
    ld_a = pltpu.async_copy(alpha_hbm, alpha_v, sem_a)
    ld_t.wait()
    ld_b.wait()
    for i in range(_BPW // _L):
        idx = t_v[pl.ds(i * _L, _L)]
        bout_v[pl.ds(i * _L, _L)] = plsc.load_gather(beta_v, [idx])
    st_b = pltpu.async_copy(bout_v, beta_out.at[pl.ds(base, _BPW)], sem_b)
    ld_a.wait()
    for i in range(_BPW // _L):
        idx = t_v[pl.ds(i * _L, _L)]
        aout_v[pl.ds(i * _L, _L)] = plsc.load_gather(alpha_v, [idx])
    st_a = pltpu.async_copy(aout_v, alpha_out.at[pl.ds(base, _BPW)], sem_a)
    st_b.wait()
    st_a.wait()


def kernel(t, beta, alpha):
    return _ddpm_gather(t.astype(jnp.int32), beta, alpha)
